# baseline (speedup 1.0000x reference)
import jax
import jax.numpy as jnp
from jax.experimental import pallas as pl
from jax.experimental.pallas import tpu as pltpu

_R = 128
_N = 32768
_B = 1024
_NB = _N // _B


def _body(u_ref, x_ref, o_ref, carry_ref):
    i = pl.program_id(0)

    @pl.when(i == 0)
    def _():
        carry_ref[...] = jnp.zeros_like(carry_ref)

    x = x_ref[...]
    o = jax.lax.dot(x, u_ref[...], preferred_element_type=jnp.float32)
    o = o + carry_ref[...]
    o_ref[...] = o
    carry_ref[...] = o[:, 0:1]


def kernel(x):
    rows = jax.lax.broadcasted_iota(jnp.int32, (_B, _B), 0)
    cols = jax.lax.broadcasted_iota(jnp.int32, (_B, _B), 1)
    u = (rows >= cols).astype(jnp.float32)
    return pl.pallas_call(
        _body,
        grid=(_NB,),
        in_specs=[
            pl.BlockSpec((_B, _B), lambda i: (0, 0)),
            pl.BlockSpec((_R, _B), lambda i: (0, _NB - 1 - i)),
        ],
        out_specs=pl.BlockSpec((_R, _B), lambda i: (0, _NB - 1 - i)),
        out_shape=jax.ShapeDtypeStruct((_R, _N), jnp.float32),
        scratch_shapes=[pltpu.VMEM((_R, 1), jnp.float32)],
        compiler_params=pltpu.CompilerParams(
            dimension_semantics=("arbitrary",),
        ),
    )(u, x)


# speedup vs baseline: 1.3062x; 1.3062x over previous
import jax
import jax.numpy as jnp
from jax.experimental import pallas as pl
from jax.experimental.pallas import tpu as pltpu

_R = 128
_N = 32768
_B = 2048
_NB = _N // _B


def _body(x_ref, o_ref, carry_ref):
    i = pl.program_id(0)

    @pl.when(i == 0)
    def _():
        carry_ref[...] = jnp.zeros_like(carry_ref)

    x = x_ref[...]
    rows = jax.lax.broadcasted_iota(jnp.int32, (_B, _B), 0)
    cols = jax.lax.broadcasted_iota(jnp.int32, (_B, _B), 1)
    u = (rows >= cols).astype(jnp.float32)
    o = jax.lax.dot(x, u, preferred_element_type=jnp.float32)
    o = o + carry_ref[...]
    o_ref[...] = o
    carry_ref[...] = o[:, 0:1]


def kernel(x):
    return pl.pallas_call(
        _body,
        grid=(_NB,),
        in_specs=[pl.BlockSpec((_R, _B), lambda i: (0, _NB - 1 - i))],
        out_specs=pl.BlockSpec((_R, _B), lambda i: (0, _NB - 1 - i)),
        out_shape=jax.ShapeDtypeStruct((_R, _N), jnp.float32),
        scratch_shapes=[pltpu.VMEM((_R, 1), jnp.float32)],
        compiler_params=pltpu.CompilerParams(
            dimension_semantics=("arbitrary",),
        ),
    )(x)
